# baseline (speedup 1.0000x reference)
import jax
import jax.numpy as jnp
from jax.experimental import pallas as pl
from jax.experimental.pallas import tpu as pltpu


def _fused_kernel(x0_ref, x1_ref, x2_ref, w0_ref, w1_ref, w2_ref,
                  b0_ref, b1_ref, b2_ref, o0_ref, o1_ref, o2_ref):
    dims = (((1,), (1,)), ((), ()))
    for x_ref, w_ref, b_ref, o_ref in (
        (x0_ref, w0_ref, b0_ref, o0_ref),
        (x1_ref, w1_ref, b1_ref, o1_ref),
        (x2_ref, w2_ref, b2_ref, o2_ref),
    ):
        o_ref[0] = jax.lax.dot_general(
            x_ref[0].astype(jnp.bfloat16),
            w_ref[...].astype(jnp.bfloat16),
            dims,
            preferred_element_type=jnp.float32,
        ) + b_ref[...]


@jax.jit
def _head(feat0, feat1, feat2, W0, b0, W1, b1, W2, b2):
    Bn = feat0.shape[0]
    shapes = [feat0.shape, feat1.shape, feat2.shape]
    O = W0.shape[0]

    xs = [
        f.transpose(0, 2, 3, 1).reshape(Bn, sy * sx, c)
        for f, (_, c, sy, sx) in zip((feat0, feat1, feat2), shapes)
    ]
    bs = [b.reshape(1, O) for b in (b0, b1, b2)]

    o0, o1, o2 = pl.pallas_call(
        _fused_kernel,
        grid=(Bn,),
        in_specs=(
            [pl.BlockSpec((1, sy * sx, c), lambda i: (i, 0, 0))
             for (_, c, sy, sx) in shapes]
            + [pl.BlockSpec((O, c), lambda i: (0, 0))
               for (_, c, _, _) in shapes]
            + [pl.BlockSpec((1, O), lambda i: (0, 0))] * 3
        ),
        out_specs=[
            pl.BlockSpec((1, sy * sx, O), lambda i: (i, 0, 0))
            for (_, _, sy, sx) in shapes
        ],
        out_shape=[
            jax.ShapeDtypeStruct((Bn, sy * sx, O), jnp.float32)
            for (_, _, sy, sx) in shapes
        ],
        compiler_params=pltpu.CompilerParams(
            vmem_limit_bytes=100 * 1024 * 1024,
            dimension_semantics=("parallel",),
        ),
    )(*xs, W0, W1, W2, *bs)

    outs = []
    for o, (_, _, sy, sx) in zip((o0, o1, o2), shapes):
        outs.append(o.reshape(Bn, sy, sx, O))
    return tuple(outs)


def kernel(feat0, feat1, feat2, W0, b0, W1, b1, W2, b2):
    return _head(feat0, feat1, feat2, W0, b0, W1, b1, W2, b2)
